# baseline (speedup 1.0000x reference)
import functools

import jax
import jax.numpy as jnp
from jax import lax
from jax.experimental import pallas as pl
from jax.experimental.pallas import tpu as pltpu
from jax.experimental.pallas import tpu_sc as plsc

EB = 128
NSUB = 16
ROWS_BLK = 128


def _seg_sum_sc(npad, epad, hh):
    nb = epad // (NSUB * EB)
    cb = 32
    nc = nb // cb
    rpt = npad // NSUB
    mesh = plsc.VectorSubcoreMesh(core_axis_name="c", subcore_axis_name="s")

    @functools.partial(
        pl.kernel,
        mesh=mesh,
        out_type=jax.ShapeDtypeStruct((2, npad, hh), jnp.float32),
        scratch_types=[
            pltpu.VMEM((cb, EB), jnp.int32),
            pltpu.VMEM((cb, EB), jnp.int32),
            pltpu.VMEM((EB, hh), jnp.float32),
            pltpu.VMEM((EB, hh), jnp.float32),
            pltpu.VMEM_SHARED((npad, hh), jnp.float32),
            pltpu.SemaphoreType.DMA,
            pltpu.SemaphoreType.DMA,
        ],
    )
    def seg(h_hbm, src_hbm, dst_hbm, out_hbm, sidx, didx, r0, r1, accum, g0, g1):
        cid = lax.axis_index("c")
        tid = lax.axis_index("s")
        hc = h_hbm.at[cid]
        bufs = (r0, r1)
        sems = (g0, g1)

        @pl.loop(0, EB)
        def _(r):
            for c in range(hh // 16):
                r0[r, pl.ds(c * 16, 16)] = jnp.zeros((16,), jnp.float32)

        for k in range(rpt // ROWS_BLK):
            pltpu.sync_copy(r0, accum.at[pl.ds(tid * rpt + k * ROWS_BLK, ROWS_BLK)])
        plsc.subcore_barrier()

        @pl.loop(0, nc)
        def _(c):
            pltpu.sync_copy(src_hbm.at[pl.ds(tid * nb + c * cb, cb)], sidx)
            pltpu.sync_copy(dst_hbm.at[pl.ds(tid * nb + c * cb, cb)], didx)
            for b in range(2):
                pltpu.async_copy(hc.at[sidx.at[b]], bufs[b], sems[b])

            @pl.loop(0, cb - 2, step=2)
            def _(j):
                for b in range(2):
                    pltpu.make_async_copy(hc.at[sidx.at[j + b]], bufs[b], sems[b]).wait()
                    pltpu.sync_copy(bufs[b], accum.at[didx.at[j + b]], add=True)
                    pltpu.async_copy(hc.at[sidx.at[j + 2 + b]], bufs[b], sems[b])

            for b in range(2):
                jl = cb - 2 + b
                pltpu.make_async_copy(hc.at[sidx.at[jl]], bufs[b], sems[b]).wait()
                pltpu.sync_copy(bufs[b], accum.at[didx.at[jl]], add=True)

        plsc.subcore_barrier()
        for k in range(rpt // ROWS_BLK):
            rr = tid * rpt + k * ROWS_BLK
            pltpu.sync_copy(accum.at[pl.ds(rr, ROWS_BLK)], r0)
            pltpu.sync_copy(r0, out_hbm.at[cid].at[pl.ds(rr, ROWS_BLK)])

    return seg


def _counts_sc(npad, epad):
    nb = epad // (NSUB * EB)
    rpt = npad // NSUB
    mesh = plsc.VectorSubcoreMesh(core_axis_name="c", subcore_axis_name="s")

    @functools.partial(
        pl.kernel,
        mesh=mesh,
        out_type=jax.ShapeDtypeStruct((2, npad), jnp.float32),
        scratch_types=[
            pltpu.VMEM((nb, EB), jnp.int32),
            pltpu.VMEM((EB,), jnp.float32),
            pltpu.VMEM((rpt,), jnp.float32),
            pltpu.VMEM_SHARED((npad,), jnp.float32),
        ],
    )
    def cnt(dsta_hbm, dstb_hbm, out_hbm, didx, ones, stage, accum):
        cid = lax.axis_index("c")
        tid = lax.axis_index("s")

        for c in range(EB // 16):
            ones[pl.ds(c * 16, 16)] = jnp.ones((16,), jnp.float32)

        @pl.loop(0, rpt // 16)
        def _(i):
            stage[pl.ds(i * 16, 16)] = jnp.zeros((16,), jnp.float32)

        pltpu.sync_copy(stage, accum.at[pl.ds(tid * rpt, rpt)])
        plsc.subcore_barrier()

        def run(dref):
            pltpu.sync_copy(dref.at[pl.ds(tid * nb, nb)], didx)

            @pl.loop(0, nb)
            def _(j):
                pltpu.sync_copy(ones, accum.at[didx.at[j]], add=True)

        @pl.when(cid == 0)
        def _():
            run(dsta_hbm)

        @pl.when(cid == 1)
        def _():
            run(dstb_hbm)

        plsc.subcore_barrier()
        pltpu.sync_copy(accum.at[pl.ds(tid * rpt, rpt)], stage)
        pltpu.sync_copy(stage, out_hbm.at[cid].at[pl.ds(tid * rpt, rpt)])

    return cnt


def _leaky(x):
    return jnp.where(x > 0, x, 0.01 * x)


def _proj_tc(x, w, b, npad, rblk=2000):
    n, d = x.shape
    h = w.shape[1]
    hh = h // 2

    def body(x_ref, w_ref, b_ref, o_ref):
        y = jnp.dot(x_ref[...], w_ref[...], preferred_element_type=jnp.float32)
        y = _leaky(y + b_ref[...])
        o_ref[0] = y[:, :hh]
        o_ref[1] = y[:, hh:]

    return pl.pallas_call(
        body,
        grid=(n // rblk,),
        in_specs=[
            pl.BlockSpec((rblk, d), lambda i: (i, 0)),
            pl.BlockSpec((d, h), lambda i: (0, 0)),
            pl.BlockSpec((1, h), lambda i: (0, 0)),
        ],
        out_specs=pl.BlockSpec((2, rblk, hh), lambda i: (0, i, 0)),
        out_shape=jax.ShapeDtypeStruct((2, npad, hh), jnp.float32),
    )(x, w, b)


def _layer_tc(aggr, cnt2, hdst, wl, bl, wr, g, beta, n, npad, act, stacked,
              rblk=2000):
    h = wl.shape[0]
    out_c = wl.shape[1]
    hh = h // 2

    def body(a_ref, c_ref, h_ref, wl_ref, bl_ref, wr_ref, g_ref, be_ref, o_ref):
        a = jnp.concatenate([a_ref[0], a_ref[1]], axis=1)
        hb = jnp.concatenate([h_ref[0], h_ref[1]], axis=1)
        mean = a / jnp.maximum(c_ref[...], 1.0)
        y = (jnp.dot(mean, wl_ref[...], preferred_element_type=jnp.float32)
             + jnp.dot(hb, wr_ref[...], preferred_element_type=jnp.float32)
             + bl_ref[...] + hb)
        mu = jnp.mean(y, axis=1, keepdims=True)
        var = jnp.mean((y - mu) * (y - mu), axis=1, keepdims=True)
        y = (y - mu) * lax.rsqrt(var + 1e-5) * g_ref[...] + be_ref[...]
        if act:
            y = _leaky(y)
        if stacked:
            o_ref[0] = y[:, :hh]
            o_ref[1] = y[:, hh:]
        else:
            o_ref[...] = y

    if stacked:
        out_spec = pl.BlockSpec((2, rblk, out_c // 2), lambda i: (0, i, 0))
        out_shape = jax.ShapeDtypeStruct((2, npad, out_c // 2), jnp.float32)
    else:
        out_spec = pl.BlockSpec((rblk, out_c), lambda i: (i, 0))
        out_shape = jax.ShapeDtypeStruct((n, out_c), jnp.float32)

    return pl.pallas_call(
        body,
        grid=(n // rblk,),
        in_specs=[
            pl.BlockSpec((2, rblk, hh), lambda i: (0, i, 0)),
            pl.BlockSpec((rblk, 1), lambda i: (i, 0)),
            pl.BlockSpec((2, rblk, hh), lambda i: (0, i, 0)),
            pl.BlockSpec((h, out_c), lambda i: (0, 0)),
            pl.BlockSpec((1, out_c), lambda i: (0, 0)),
            pl.BlockSpec((h, out_c), lambda i: (0, 0)),
            pl.BlockSpec((1, out_c), lambda i: (0, 0)),
            pl.BlockSpec((1, out_c), lambda i: (0, 0)),
        ],
        out_specs=out_spec,
        out_shape=out_shape,
    )(aggr, cnt2, hdst, wl, bl, wr, g, beta)


def kernel(x_gene, x_cell, edge_index_g2c, edge_index_c2g, params):
    p = params
    n, d_in = x_gene.shape
    h = p["in_gene_W"].shape[1]
    e = edge_index_g2c.shape[1]

    npad = ((n + NSUB * ROWS_BLK - 1) // (NSUB * ROWS_BLK)) * (NSUB * ROWS_BLK)
    egrain = NSUB * EB * 8
    epad = ((e + egrain - 1) // egrain) * egrain

    padn = epad - e
    pad_src = jnp.zeros((padn,), jnp.int32)
    pad_dst = n + jnp.arange(padn, dtype=jnp.int32) % max(npad - n, 1)

    def prep(ei):
        src = jnp.concatenate([ei[0], pad_src]).reshape(-1, EB)
        dst = jnp.concatenate([ei[1], pad_dst]).reshape(-1, EB)
        return src, dst

    src_g2c, dst_g2c = prep(edge_index_g2c)
    src_c2g, dst_c2g = prep(edge_index_c2g)

    seg = _seg_sum_sc(npad, epad, h // 2)
    cnts = _counts_sc(npad, epad)(dst_g2c, dst_c2g)
    cnt_cell = cnts[0][:, None]
    cnt_gene = cnts[1][:, None]

    hg = _proj_tc(x_gene, p["in_gene_W"], p["in_gene_b"][None, :], npad)
    hc = _proj_tc(x_cell, p["in_cell_W"], p["in_cell_b"][None, :], npad)

    num_layers = 2
    for l in range(num_layers):
        aggr_cell = seg(hg, src_g2c, dst_g2c)
        aggr_gene = seg(hc, src_c2g, dst_c2g)
        last = l == num_layers - 1
        hc_new = _layer_tc(
            aggr_cell, cnt_cell, hc,
            p["l%d_g2c_Wl" % l], p["l%d_g2c_bl" % l][None, :], p["l%d_g2c_Wr" % l],
            p["l%d_cell_g" % l][None, :], p["l%d_cell_b" % l][None, :],
            n, npad, act=not last, stacked=not last)
        hg_new = _layer_tc(
            aggr_gene, cnt_gene, hg,
            p["l%d_c2g_Wl" % l], p["l%d_c2g_bl" % l][None, :], p["l%d_c2g_Wr" % l],
            p["l%d_gene_g" % l][None, :], p["l%d_gene_b" % l][None, :],
            n, npad, act=not last, stacked=not last)
        hg, hc = hg_new, hc_new

    return (hg, hc)


# speedup vs baseline: 1.4107x; 1.4107x over previous
import functools

import jax
import jax.numpy as jnp
from jax import lax
from jax.experimental import pallas as pl
from jax.experimental.pallas import tpu as pltpu
from jax.experimental.pallas import tpu_sc as plsc

EB = 128
NSUB = 16
ROWS_BLK = 128


def _seg_sum_sc(npad, epad, hh):
    nb = epad // (NSUB * EB)
    cb = 32
    nc = nb // cb
    rpt = npad // NSUB
    hq = hh // 2
    mesh = plsc.VectorSubcoreMesh(core_axis_name="c", subcore_axis_name="s")

    @functools.partial(
        pl.kernel,
        mesh=mesh,
        out_type=jax.ShapeDtypeStruct((2, npad, hh), jnp.float32),
        compiler_params=pltpu.CompilerParams(use_tc_tiling_on_sc=False),
        scratch_types=[
            pltpu.VMEM((cb, EB), jnp.int32),
            pltpu.VMEM((cb, EB), jnp.int32),
            pltpu.VMEM((EB, hq), jnp.float32),
            pltpu.VMEM_SHARED((npad, hq), jnp.float32),
            pltpu.VMEM_SHARED((npad, hq), jnp.float32),
            pltpu.SemaphoreType.DMA,
        ],
    )
    def seg(h_hbm, src_hbm, dst_hbm, out_hbm, sidx, didx, rows, table, accum, g0):
        cid = lax.axis_index("c")
        tid = lax.axis_index("s")

        @pl.loop(0, EB)
        def _(r):
            for c in range(hq // 16):
                rows[r, pl.ds(c * 16, 16)] = jnp.zeros((16,), jnp.float32)

        for p in range(2):
            cbase = p * hq
            pltpu.sync_copy(
                h_hbm.at[cid, pl.ds(tid * rpt, rpt), pl.ds(cbase, hq)],
                table.at[pl.ds(tid * rpt, rpt)])
            for k in range(rpt // ROWS_BLK):
                pltpu.sync_copy(
                    rows, accum.at[pl.ds(tid * rpt + k * ROWS_BLK, ROWS_BLK)])
            plsc.subcore_barrier()

            @pl.loop(0, nc)
            def _(c):
                pltpu.sync_copy(src_hbm.at[pl.ds(tid * nb + c * cb, cb)], sidx)
                pltpu.sync_copy(dst_hbm.at[pl.ds(tid * nb + c * cb, cb)], didx)

                @pl.loop(0, cb)
                def _(j):
                    pltpu.async_copy(table.at[sidx.at[j]], rows, g0).wait()
                    pltpu.sync_copy(rows, accum.at[didx.at[j]], add=True)

            plsc.subcore_barrier()
            for k in range(rpt // ROWS_BLK):
                rr = tid * rpt + k * ROWS_BLK
                pltpu.sync_copy(accum.at[pl.ds(rr, ROWS_BLK)], rows)
                pltpu.sync_copy(
                    rows, out_hbm.at[cid, pl.ds(rr, ROWS_BLK), pl.ds(cbase, hq)])
            if p == 0:
                plsc.subcore_barrier()

                @pl.loop(0, EB)
                def _(r):
                    for c in range(hq // 16):
                        rows[r, pl.ds(c * 16, 16)] = jnp.zeros((16,), jnp.float32)

    return seg


def _counts_sc(npad, epad):
    nb = epad // (NSUB * EB)
    rpt = npad // NSUB
    mesh = plsc.VectorSubcoreMesh(core_axis_name="c", subcore_axis_name="s")

    @functools.partial(
        pl.kernel,
        mesh=mesh,
        out_type=jax.ShapeDtypeStruct((2, npad), jnp.float32),
        scratch_types=[
            pltpu.VMEM((nb, EB), jnp.int32),
            pltpu.VMEM((EB,), jnp.float32),
            pltpu.VMEM((rpt,), jnp.float32),
            pltpu.VMEM_SHARED((npad,), jnp.float32),
        ],
    )
    def cnt(dsta_hbm, dstb_hbm, out_hbm, didx, ones, stage, accum):
        cid = lax.axis_index("c")
        tid = lax.axis_index("s")

        for c in range(EB // 16):
            ones[pl.ds(c * 16, 16)] = jnp.ones((16,), jnp.float32)

        @pl.loop(0, rpt // 16)
        def _(i):
            stage[pl.ds(i * 16, 16)] = jnp.zeros((16,), jnp.float32)

        pltpu.sync_copy(stage, accum.at[pl.ds(tid * rpt, rpt)])
        plsc.subcore_barrier()

        def run(dref):
            pltpu.sync_copy(dref.at[pl.ds(tid * nb, nb)], didx)

            @pl.loop(0, nb)
            def _(j):
                pltpu.sync_copy(ones, accum.at[didx.at[j]], add=True)

        @pl.when(cid == 0)
        def _():
            run(dsta_hbm)

        @pl.when(cid == 1)
        def _():
            run(dstb_hbm)

        plsc.subcore_barrier()
        pltpu.sync_copy(accum.at[pl.ds(tid * rpt, rpt)], stage)
        pltpu.sync_copy(stage, out_hbm.at[cid].at[pl.ds(tid * rpt, rpt)])

    return cnt


def _leaky(x):
    return jnp.where(x > 0, x, 0.01 * x)


def _proj_tc(x, w, b, npad, rblk=2000):
    n, d = x.shape
    h = w.shape[1]
    hh = h // 2

    def body(x_ref, w_ref, b_ref, o_ref):
        y = jnp.dot(x_ref[...], w_ref[...], preferred_element_type=jnp.float32)
        y = _leaky(y + b_ref[...])
        o_ref[0] = y[:, :hh]
        o_ref[1] = y[:, hh:]

    return pl.pallas_call(
        body,
        grid=(n // rblk,),
        in_specs=[
            pl.BlockSpec((rblk, d), lambda i: (i, 0)),
            pl.BlockSpec((d, h), lambda i: (0, 0)),
            pl.BlockSpec((1, h), lambda i: (0, 0)),
        ],
        out_specs=pl.BlockSpec((2, rblk, hh), lambda i: (0, i, 0)),
        out_shape=jax.ShapeDtypeStruct((2, npad, hh), jnp.float32),
    )(x, w, b)


def _layer_tc(aggr, cnt2, hdst, wl, bl, wr, g, beta, n, npad, act, stacked,
              rblk=2000):
    h = wl.shape[0]
    out_c = wl.shape[1]
    hh = h // 2

    def body(a_ref, c_ref, h_ref, wl_ref, bl_ref, wr_ref, g_ref, be_ref, o_ref):
        a = jnp.concatenate([a_ref[0], a_ref[1]], axis=1)
        hb = jnp.concatenate([h_ref[0], h_ref[1]], axis=1)
        mean = a / jnp.maximum(c_ref[...], 1.0)
        y = (jnp.dot(mean, wl_ref[...], preferred_element_type=jnp.float32)
             + jnp.dot(hb, wr_ref[...], preferred_element_type=jnp.float32)
             + bl_ref[...] + hb)
        mu = jnp.mean(y, axis=1, keepdims=True)
        var = jnp.mean((y - mu) * (y - mu), axis=1, keepdims=True)
        y = (y - mu) * lax.rsqrt(var + 1e-5) * g_ref[...] + be_ref[...]
        if act:
            y = _leaky(y)
        if stacked:
            o_ref[0] = y[:, :hh]
            o_ref[1] = y[:, hh:]
        else:
            o_ref[...] = y

    if stacked:
        out_spec = pl.BlockSpec((2, rblk, out_c // 2), lambda i: (0, i, 0))
        out_shape = jax.ShapeDtypeStruct((2, npad, out_c // 2), jnp.float32)
    else:
        out_spec = pl.BlockSpec((rblk, out_c), lambda i: (i, 0))
        out_shape = jax.ShapeDtypeStruct((n, out_c), jnp.float32)

    return pl.pallas_call(
        body,
        grid=(n // rblk,),
        in_specs=[
            pl.BlockSpec((2, rblk, hh), lambda i: (0, i, 0)),
            pl.BlockSpec((rblk, 1), lambda i: (i, 0)),
            pl.BlockSpec((2, rblk, hh), lambda i: (0, i, 0)),
            pl.BlockSpec((h, out_c), lambda i: (0, 0)),
            pl.BlockSpec((1, out_c), lambda i: (0, 0)),
            pl.BlockSpec((h, out_c), lambda i: (0, 0)),
            pl.BlockSpec((1, out_c), lambda i: (0, 0)),
            pl.BlockSpec((1, out_c), lambda i: (0, 0)),
        ],
        out_specs=out_spec,
        out_shape=out_shape,
    )(aggr, cnt2, hdst, wl, bl, wr, g, beta)


def kernel(x_gene, x_cell, edge_index_g2c, edge_index_c2g, params):
    p = params
    n, d_in = x_gene.shape
    h = p["in_gene_W"].shape[1]
    e = edge_index_g2c.shape[1]

    npad = ((n + NSUB * ROWS_BLK - 1) // (NSUB * ROWS_BLK)) * (NSUB * ROWS_BLK)
    egrain = NSUB * EB * 8
    epad = ((e + egrain - 1) // egrain) * egrain

    padn = epad - e
    pad_src = jnp.zeros((padn,), jnp.int32)
    pad_dst = n + jnp.arange(padn, dtype=jnp.int32) % max(npad - n, 1)

    def prep(ei):
        src = jnp.concatenate([ei[0], pad_src]).reshape(-1, EB)
        dst = jnp.concatenate([ei[1], pad_dst]).reshape(-1, EB)
        return src, dst

    src_g2c, dst_g2c = prep(edge_index_g2c)
    src_c2g, dst_c2g = prep(edge_index_c2g)

    seg = _seg_sum_sc(npad, epad, h // 2)
    cnts = _counts_sc(npad, epad)(dst_g2c, dst_c2g)
    cnt_cell = cnts[0][:, None]
    cnt_gene = cnts[1][:, None]

    hg = _proj_tc(x_gene, p["in_gene_W"], p["in_gene_b"][None, :], npad)
    hc = _proj_tc(x_cell, p["in_cell_W"], p["in_cell_b"][None, :], npad)

    num_layers = 2
    for l in range(num_layers):
        aggr_cell = seg(hg, src_g2c, dst_g2c)
        aggr_gene = seg(hc, src_c2g, dst_c2g)
        last = l == num_layers - 1
        hc_new = _layer_tc(
            aggr_cell, cnt_cell, hc,
            p["l%d_g2c_Wl" % l], p["l%d_g2c_bl" % l][None, :], p["l%d_g2c_Wr" % l],
            p["l%d_cell_g" % l][None, :], p["l%d_cell_b" % l][None, :],
            n, npad, act=not last, stacked=not last)
        hg_new = _layer_tc(
            aggr_gene, cnt_gene, hg,
            p["l%d_c2g_Wl" % l], p["l%d_c2g_bl" % l][None, :], p["l%d_c2g_Wr" % l],
            p["l%d_gene_g" % l][None, :], p["l%d_gene_b" % l][None, :],
            n, npad, act=not last, stacked=not last)
        hg, hc = hg_new, hc_new

    return (hg, hc)


# speedup vs baseline: 1.8786x; 1.3317x over previous
import functools

import jax
import jax.numpy as jnp
from jax import lax
from jax.experimental import pallas as pl
from jax.experimental.pallas import tpu as pltpu
from jax.experimental.pallas import tpu_sc as plsc

EB = 128
NSUB = 16
ROWS_BLK = 128


def _seg_sum_sc(npad, epad, hh):
    nb = epad // (NSUB * EB)
    cb = 32
    nc = nb // cb
    rpt = npad // NSUB
    hq = hh // 2
    mesh = plsc.VectorSubcoreMesh(core_axis_name="c", subcore_axis_name="s")

    @functools.partial(
        pl.kernel,
        mesh=mesh,
        out_type=jax.ShapeDtypeStruct((2, npad, hh), jnp.float32),
        compiler_params=pltpu.CompilerParams(use_tc_tiling_on_sc=False),
        scratch_types=[
            pltpu.VMEM((cb, EB), jnp.int32),
            pltpu.VMEM((cb, EB), jnp.int32),
            pltpu.VMEM((EB, hq), jnp.float32),
            pltpu.VMEM((EB, hq), jnp.float32),
            pltpu.VMEM((ROWS_BLK, hq), jnp.float32),
            pltpu.VMEM_SHARED((npad, hq), jnp.float32),
            pltpu.VMEM_SHARED((npad, hq), jnp.float32),
            pltpu.SemaphoreType.DMA,
            pltpu.SemaphoreType.DMA,
        ],
    )
    def seg(h_hbm, src_hbm, dst_hbm, out_hbm, sidx, didx, rows0, rows1, zbuf,
            table, accum, g0, g1):
        cid = lax.axis_index("c")
        tid = lax.axis_index("s")

        @pl.loop(0, ROWS_BLK)
        def _(r):
            for c in range(hq // 16):
                zbuf[r, pl.ds(c * 16, 16)] = jnp.zeros((16,), jnp.float32)

        for p in range(2):
            cbase = p * hq
            pltpu.sync_copy(
                h_hbm.at[cid, pl.ds(tid * rpt, rpt), pl.ds(cbase, hq)],
                table.at[pl.ds(tid * rpt, rpt)])
            for k in range(rpt // ROWS_BLK):
                pltpu.sync_copy(
                    zbuf, accum.at[pl.ds(tid * rpt + k * ROWS_BLK, ROWS_BLK)])
            plsc.subcore_barrier()

            @pl.loop(0, nc)
            def _(c):
                pltpu.sync_copy(src_hbm.at[pl.ds(tid * nb + c * cb, cb)], sidx)
                pltpu.sync_copy(dst_hbm.at[pl.ds(tid * nb + c * cb, cb)], didx)

                pltpu.async_copy(table.at[sidx.at[0]], rows0, g0)

                @pl.loop(0, cb, step=2)
                def _(j):
                    pltpu.async_copy(table.at[sidx.at[j + 1]], rows1, g1)
                    pltpu.make_async_copy(table.at[sidx.at[j]], rows0, g0).wait()
                    pltpu.sync_copy(rows0, accum.at[didx.at[j]], add=True)

                    @pl.when(j + 2 < cb)
                    def _():
                        pltpu.async_copy(table.at[sidx.at[j + 2]], rows0, g0)

                    pltpu.make_async_copy(
                        table.at[sidx.at[j + 1]], rows1, g1).wait()
                    pltpu.sync_copy(rows1, accum.at[didx.at[j + 1]], add=True)

            plsc.subcore_barrier()
            for k in range(rpt // ROWS_BLK):
                rr = tid * rpt + k * ROWS_BLK
                pltpu.sync_copy(accum.at[pl.ds(rr, ROWS_BLK)], rows0)
                pltpu.sync_copy(
                    rows0, out_hbm.at[cid, pl.ds(rr, ROWS_BLK), pl.ds(cbase, hq)])
            if p == 0:
                plsc.subcore_barrier()

    return seg


def _counts_sc(npad, epad):
    nb = epad // (NSUB * EB)
    rpt = npad // NSUB
    mesh = plsc.VectorSubcoreMesh(core_axis_name="c", subcore_axis_name="s")

    @functools.partial(
        pl.kernel,
        mesh=mesh,
        out_type=jax.ShapeDtypeStruct((2, npad), jnp.float32),
        scratch_types=[
            pltpu.VMEM((nb, EB), jnp.int32),
            pltpu.VMEM((EB,), jnp.float32),
            pltpu.VMEM((rpt,), jnp.float32),
            pltpu.VMEM_SHARED((npad,), jnp.float32),
        ],
    )
    def cnt(dsta_hbm, dstb_hbm, out_hbm, didx, ones, stage, accum):
        cid = lax.axis_index("c")
        tid = lax.axis_index("s")

        for c in range(EB // 16):
            ones[pl.ds(c * 16, 16)] = jnp.ones((16,), jnp.float32)

        @pl.loop(0, rpt // 16)
        def _(i):
            stage[pl.ds(i * 16, 16)] = jnp.zeros((16,), jnp.float32)

        pltpu.sync_copy(stage, accum.at[pl.ds(tid * rpt, rpt)])
        plsc.subcore_barrier()

        def run(dref):
            pltpu.sync_copy(dref.at[pl.ds(tid * nb, nb)], didx)

            @pl.loop(0, nb)
            def _(j):
                pltpu.sync_copy(ones, accum.at[didx.at[j]], add=True)

        @pl.when(cid == 0)
        def _():
            run(dsta_hbm)

        @pl.when(cid == 1)
        def _():
            run(dstb_hbm)

        plsc.subcore_barrier()
        pltpu.sync_copy(accum.at[pl.ds(tid * rpt, rpt)], stage)
        pltpu.sync_copy(stage, out_hbm.at[cid].at[pl.ds(tid * rpt, rpt)])

    return cnt


def _leaky(x):
    return jnp.where(x > 0, x, 0.01 * x)


def _proj_tc(x, w, b, npad, rblk=2000):
    n, d = x.shape
    h = w.shape[1]
    hh = h // 2

    def body(x_ref, w_ref, b_ref, o_ref):
        y = jnp.dot(x_ref[...], w_ref[...], preferred_element_type=jnp.float32)
        y = _leaky(y + b_ref[...])
        o_ref[0] = y[:, :hh]
        o_ref[1] = y[:, hh:]

    return pl.pallas_call(
        body,
        grid=(n // rblk,),
        in_specs=[
            pl.BlockSpec((rblk, d), lambda i: (i, 0)),
            pl.BlockSpec((d, h), lambda i: (0, 0)),
            pl.BlockSpec((1, h), lambda i: (0, 0)),
        ],
        out_specs=pl.BlockSpec((2, rblk, hh), lambda i: (0, i, 0)),
        out_shape=jax.ShapeDtypeStruct((2, npad, hh), jnp.float32),
    )(x, w, b)


def _layer_tc(aggr, cnt2, hdst, wl, bl, wr, g, beta, n, npad, act, stacked,
              rblk=2000):
    h = wl.shape[0]
    out_c = wl.shape[1]
    hh = h // 2

    def body(a_ref, c_ref, h_ref, wl_ref, bl_ref, wr_ref, g_ref, be_ref, o_ref):
        a = jnp.concatenate([a_ref[0], a_ref[1]], axis=1)
        hb = jnp.concatenate([h_ref[0], h_ref[1]], axis=1)
        mean = a / jnp.maximum(c_ref[...], 1.0)
        y = (jnp.dot(mean, wl_ref[...], preferred_element_type=jnp.float32)
             + jnp.dot(hb, wr_ref[...], preferred_element_type=jnp.float32)
             + bl_ref[...] + hb)
        mu = jnp.mean(y, axis=1, keepdims=True)
        var = jnp.mean((y - mu) * (y - mu), axis=1, keepdims=True)
        y = (y - mu) * lax.rsqrt(var + 1e-5) * g_ref[...] + be_ref[...]
        if act:
            y = _leaky(y)
        if stacked:
            o_ref[0] = y[:, :hh]
            o_ref[1] = y[:, hh:]
        else:
            o_ref[...] = y

    if stacked:
        out_spec = pl.BlockSpec((2, rblk, out_c // 2), lambda i: (0, i, 0))
        out_shape = jax.ShapeDtypeStruct((2, npad, out_c // 2), jnp.float32)
    else:
        out_spec = pl.BlockSpec((rblk, out_c), lambda i: (i, 0))
        out_shape = jax.ShapeDtypeStruct((n, out_c), jnp.float32)

    return pl.pallas_call(
        body,
        grid=(n // rblk,),
        in_specs=[
            pl.BlockSpec((2, rblk, hh), lambda i: (0, i, 0)),
            pl.BlockSpec((rblk, 1), lambda i: (i, 0)),
            pl.BlockSpec((2, rblk, hh), lambda i: (0, i, 0)),
            pl.BlockSpec((h, out_c), lambda i: (0, 0)),
            pl.BlockSpec((1, out_c), lambda i: (0, 0)),
            pl.BlockSpec((h, out_c), lambda i: (0, 0)),
            pl.BlockSpec((1, out_c), lambda i: (0, 0)),
            pl.BlockSpec((1, out_c), lambda i: (0, 0)),
        ],
        out_specs=out_spec,
        out_shape=out_shape,
    )(aggr, cnt2, hdst, wl, bl, wr, g, beta)


def kernel(x_gene, x_cell, edge_index_g2c, edge_index_c2g, params):
    p = params
    n, d_in = x_gene.shape
    h = p["in_gene_W"].shape[1]
    e = edge_index_g2c.shape[1]

    npad = ((n + NSUB * ROWS_BLK - 1) // (NSUB * ROWS_BLK)) * (NSUB * ROWS_BLK)
    egrain = NSUB * EB * 8
    epad = ((e + egrain - 1) // egrain) * egrain

    padn = epad - e
    pad_src = jnp.zeros((padn,), jnp.int32)
    pad_dst = n + jnp.arange(padn, dtype=jnp.int32) % max(npad - n, 1)

    def prep(ei):
        src = jnp.concatenate([ei[0], pad_src]).reshape(-1, EB)
        dst = jnp.concatenate([ei[1], pad_dst]).reshape(-1, EB)
        return src, dst

    src_g2c, dst_g2c = prep(edge_index_g2c)
    src_c2g, dst_c2g = prep(edge_index_c2g)

    seg = _seg_sum_sc(npad, epad, h // 2)
    cnts = _counts_sc(npad, epad)(dst_g2c, dst_c2g)
    cnt_cell = cnts[0][:, None]
    cnt_gene = cnts[1][:, None]

    hg = _proj_tc(x_gene, p["in_gene_W"], p["in_gene_b"][None, :], npad)
    hc = _proj_tc(x_cell, p["in_cell_W"], p["in_cell_b"][None, :], npad)

    num_layers = 2
    for l in range(num_layers):
        aggr_cell = seg(hg, src_g2c, dst_g2c)
        aggr_gene = seg(hc, src_c2g, dst_c2g)
        last = l == num_layers - 1
        hc_new = _layer_tc(
            aggr_cell, cnt_cell, hc,
            p["l%d_g2c_Wl" % l], p["l%d_g2c_bl" % l][None, :], p["l%d_g2c_Wr" % l],
            p["l%d_cell_g" % l][None, :], p["l%d_cell_b" % l][None, :],
            n, npad, act=not last, stacked=not last)
        hg_new = _layer_tc(
            aggr_gene, cnt_gene, hg,
            p["l%d_c2g_Wl" % l], p["l%d_c2g_bl" % l][None, :], p["l%d_c2g_Wr" % l],
            p["l%d_gene_g" % l][None, :], p["l%d_gene_b" % l][None, :],
            n, npad, act=not last, stacked=not last)
        hg, hc = hg_new, hc_new

    return (hg, hc)


# speedup vs baseline: 3.1901x; 1.6981x over previous
import functools

import jax
import jax.numpy as jnp
from jax import lax
from jax.experimental import pallas as pl
from jax.experimental.pallas import tpu as pltpu
from jax.experimental.pallas import tpu_sc as plsc

EB = 128
NSUB = 16
ROWS_BLK = 128


def _seg_sum_sc(npad, epad, hh):
    nb = epad // (NSUB * EB)
    cb = 32
    nc = nb // cb
    rpt = npad // NSUB
    hq = hh // 2
    mesh = plsc.VectorSubcoreMesh(core_axis_name="c", subcore_axis_name="s")

    @functools.partial(
        pl.kernel,
        mesh=mesh,
        out_type=jax.ShapeDtypeStruct((2, npad, hh), jnp.bfloat16),
        compiler_params=pltpu.CompilerParams(use_tc_tiling_on_sc=False),
        scratch_types=[
            pltpu.VMEM((cb, EB), jnp.int32),
            pltpu.VMEM((cb, EB), jnp.int32),
            pltpu.VMEM((EB, hq), jnp.bfloat16),
            pltpu.VMEM((EB, hq), jnp.bfloat16),
            pltpu.VMEM_SHARED((npad, hq), jnp.bfloat16),
            pltpu.VMEM_SHARED((npad, hq), jnp.bfloat16),
            pltpu.SemaphoreType.DMA,
            pltpu.SemaphoreType.DMA,
        ],
    )
    def seg(h_hbm, src_hbm, dst_hbm, zeros_hbm, out_hbm, sidx, didx,
            rows0, rows1, table, accum, g0, g1):
        cid = lax.axis_index("c")
        tid = lax.axis_index("s")

        for p in range(2):
            cbase = p * hq
            pltpu.sync_copy(
                h_hbm.at[cid, pl.ds(tid * rpt, rpt), pl.ds(cbase, hq)],
                table.at[pl.ds(tid * rpt, rpt)])
            pltpu.sync_copy(zeros_hbm, accum.at[pl.ds(tid * rpt, rpt)])
            plsc.subcore_barrier()

            @pl.loop(0, nc)
            def _(c):
                pltpu.sync_copy(src_hbm.at[pl.ds(tid * nb + c * cb, cb)], sidx)
                pltpu.sync_copy(dst_hbm.at[pl.ds(tid * nb + c * cb, cb)], didx)

                pltpu.async_copy(table.at[sidx.at[0]], rows0, g0)

                @pl.loop(0, cb, step=2)
                def _(j):
                    pltpu.async_copy(table.at[sidx.at[j + 1]], rows1, g1)
                    pltpu.make_async_copy(table.at[sidx.at[j]], rows0, g0).wait()
                    pltpu.sync_copy(rows0, accum.at[didx.at[j]], add=True)

                    @pl.when(j + 2 < cb)
                    def _():
                        pltpu.async_copy(table.at[sidx.at[j + 2]], rows0, g0)

                    pltpu.make_async_copy(
                        table.at[sidx.at[j + 1]], rows1, g1).wait()
                    pltpu.sync_copy(rows1, accum.at[didx.at[j + 1]], add=True)

            plsc.subcore_barrier()
            for k in range(rpt // ROWS_BLK):
                rr = tid * rpt + k * ROWS_BLK
                pltpu.sync_copy(accum.at[pl.ds(rr, ROWS_BLK)], rows0)
                pltpu.sync_copy(
                    rows0, out_hbm.at[cid, pl.ds(rr, ROWS_BLK), pl.ds(cbase, hq)])
            if p == 0:
                plsc.subcore_barrier()

    return seg


def _counts_sc(npad, epad):
    nb = epad // (NSUB * EB)
    rpt = npad // NSUB
    mesh = plsc.VectorSubcoreMesh(core_axis_name="c", subcore_axis_name="s")

    @functools.partial(
        pl.kernel,
        mesh=mesh,
        out_type=jax.ShapeDtypeStruct((2, npad), jnp.float32),
        scratch_types=[
            pltpu.VMEM((nb, EB), jnp.int32),
            pltpu.VMEM((EB,), jnp.float32),
            pltpu.VMEM((rpt,), jnp.float32),
            pltpu.VMEM_SHARED((npad,), jnp.float32),
        ],
    )
    def cnt(dsta_hbm, dstb_hbm, out_hbm, didx, ones, stage, accum):
        cid = lax.axis_index("c")
        tid = lax.axis_index("s")

        for c in range(EB // 16):
            ones[pl.ds(c * 16, 16)] = jnp.ones((16,), jnp.float32)

        @pl.loop(0, rpt // 16)
        def _(i):
            stage[pl.ds(i * 16, 16)] = jnp.zeros((16,), jnp.float32)

        pltpu.sync_copy(stage, accum.at[pl.ds(tid * rpt, rpt)])
        plsc.subcore_barrier()

        def run(dref):
            pltpu.sync_copy(dref.at[pl.ds(tid * nb, nb)], didx)

            @pl.loop(0, nb)
            def _(j):
                pltpu.sync_copy(ones, accum.at[didx.at[j]], add=True)

        @pl.when(cid == 0)
        def _():
            run(dsta_hbm)

        @pl.when(cid == 1)
        def _():
            run(dstb_hbm)

        plsc.subcore_barrier()
        pltpu.sync_copy(accum.at[pl.ds(tid * rpt, rpt)], stage)
        pltpu.sync_copy(stage, out_hbm.at[cid].at[pl.ds(tid * rpt, rpt)])

    return cnt


def _leaky(x):
    return jnp.where(x > 0, x, 0.01 * x)


def _proj_tc(x, w, b, npad, rblk=2000):
    n, d = x.shape
    h = w.shape[1]
    hh = h // 2

    def body(x_ref, w_ref, b_ref, o_ref, ob_ref):
        y = jnp.dot(x_ref[...], w_ref[...], preferred_element_type=jnp.float32)
        y = _leaky(y + b_ref[...])
        o_ref[0] = y[:, :hh]
        o_ref[1] = y[:, hh:]
        yb = y.astype(jnp.bfloat16)
        ob_ref[0] = yb[:, :hh]
        ob_ref[1] = yb[:, hh:]

    return pl.pallas_call(
        body,
        grid=(n // rblk,),
        in_specs=[
            pl.BlockSpec((rblk, d), lambda i: (i, 0)),
            pl.BlockSpec((d, h), lambda i: (0, 0)),
            pl.BlockSpec((1, h), lambda i: (0, 0)),
        ],
        out_specs=[
            pl.BlockSpec((2, rblk, hh), lambda i: (0, i, 0)),
            pl.BlockSpec((2, rblk, hh), lambda i: (0, i, 0)),
        ],
        out_shape=[
            jax.ShapeDtypeStruct((2, npad, hh), jnp.float32),
            jax.ShapeDtypeStruct((2, npad, hh), jnp.bfloat16),
        ],
    )(x, w, b)


def _layer_tc(aggr, cnt2, hdst, wl, bl, wr, g, beta, n, npad, act, stacked,
              rblk=2000):
    h = wl.shape[0]
    out_c = wl.shape[1]
    hh = h // 2

    def body(a_ref, c_ref, h_ref, wl_ref, bl_ref, wr_ref, g_ref, be_ref, *o_refs):
        a = jnp.concatenate([a_ref[0], a_ref[1]], axis=1).astype(jnp.float32)
        hb = jnp.concatenate([h_ref[0], h_ref[1]], axis=1)
        mean = a / jnp.maximum(c_ref[...], 1.0)
        y = (jnp.dot(mean, wl_ref[...], preferred_element_type=jnp.float32)
             + jnp.dot(hb, wr_ref[...], preferred_element_type=jnp.float32)
             + bl_ref[...] + hb)
        mu = jnp.mean(y, axis=1, keepdims=True)
        var = jnp.mean((y - mu) * (y - mu), axis=1, keepdims=True)
        y = (y - mu) * lax.rsqrt(var + 1e-5) * g_ref[...] + be_ref[...]
        if act:
            y = _leaky(y)
        if stacked:
            o_refs[0][0] = y[:, :hh]
            o_refs[0][1] = y[:, hh:]
            yb = y.astype(jnp.bfloat16)
            o_refs[1][0] = yb[:, :hh]
            o_refs[1][1] = yb[:, hh:]
        else:
            o_refs[0][...] = y

    if stacked:
        out_spec = [
            pl.BlockSpec((2, rblk, out_c // 2), lambda i: (0, i, 0)),
            pl.BlockSpec((2, rblk, out_c // 2), lambda i: (0, i, 0)),
        ]
        out_shape = [
            jax.ShapeDtypeStruct((2, npad, out_c // 2), jnp.float32),
            jax.ShapeDtypeStruct((2, npad, out_c // 2), jnp.bfloat16),
        ]
    else:
        out_spec = pl.BlockSpec((rblk, out_c), lambda i: (i, 0))
        out_shape = jax.ShapeDtypeStruct((n, out_c), jnp.float32)

    return pl.pallas_call(
        body,
        grid=(n // rblk,),
        in_specs=[
            pl.BlockSpec((2, rblk, hh), lambda i: (0, i, 0)),
            pl.BlockSpec((rblk, 1), lambda i: (i, 0)),
            pl.BlockSpec((2, rblk, hh), lambda i: (0, i, 0)),
            pl.BlockSpec((h, out_c), lambda i: (0, 0)),
            pl.BlockSpec((1, out_c), lambda i: (0, 0)),
            pl.BlockSpec((h, out_c), lambda i: (0, 0)),
            pl.BlockSpec((1, out_c), lambda i: (0, 0)),
            pl.BlockSpec((1, out_c), lambda i: (0, 0)),
        ],
        out_specs=out_spec,
        out_shape=out_shape,
    )(aggr, cnt2, hdst, wl, bl, wr, g, beta)


def kernel(x_gene, x_cell, edge_index_g2c, edge_index_c2g, params):
    p = params
    n, d_in = x_gene.shape
    h = p["in_gene_W"].shape[1]
    e = edge_index_g2c.shape[1]

    npad = ((n + NSUB * ROWS_BLK - 1) // (NSUB * ROWS_BLK)) * (NSUB * ROWS_BLK)
    egrain = NSUB * EB * 8
    epad = ((e + egrain - 1) // egrain) * egrain

    padn = epad - e
    pad_src = jnp.zeros((padn,), jnp.int32)
    pad_dst = n + jnp.arange(padn, dtype=jnp.int32) % max(npad - n, 1)

    def prep(ei):
        src = jnp.concatenate([ei[0], pad_src]).reshape(-1, EB)
        dst = jnp.concatenate([ei[1], pad_dst]).reshape(-1, EB)
        return src, dst

    src_g2c, dst_g2c = prep(edge_index_g2c)
    src_c2g, dst_c2g = prep(edge_index_c2g)

    seg = _seg_sum_sc(npad, epad, h // 2)
    zeros_rows = jnp.zeros((npad // NSUB, h // 4), jnp.bfloat16)
    cnts = _counts_sc(npad, epad)(dst_g2c, dst_c2g)
    cnt_cell = cnts[0][:, None]
    cnt_gene = cnts[1][:, None]

    hg, hgb = _proj_tc(x_gene, p["in_gene_W"], p["in_gene_b"][None, :], npad)
    hc, hcb = _proj_tc(x_cell, p["in_cell_W"], p["in_cell_b"][None, :], npad)

    num_layers = 2
    for l in range(num_layers):
        aggr_cell = seg(hgb, src_g2c, dst_g2c, zeros_rows)
        aggr_gene = seg(hcb, src_c2g, dst_c2g, zeros_rows)
        last = l == num_layers - 1
        out_cell = _layer_tc(
            aggr_cell, cnt_cell, hc,
            p["l%d_g2c_Wl" % l], p["l%d_g2c_bl" % l][None, :], p["l%d_g2c_Wr" % l],
            p["l%d_cell_g" % l][None, :], p["l%d_cell_b" % l][None, :],
            n, npad, act=not last, stacked=not last)
        out_gene = _layer_tc(
            aggr_gene, cnt_gene, hg,
            p["l%d_c2g_Wl" % l], p["l%d_c2g_bl" % l][None, :], p["l%d_c2g_Wr" % l],
            p["l%d_gene_g" % l][None, :], p["l%d_gene_b" % l][None, :],
            n, npad, act=not last, stacked=not last)
        if last:
            hg, hc = out_gene, out_cell
        else:
            (hg, hgb), (hc, hcb) = out_gene, out_cell

    return (hg, hc)
